# baseline (speedup 1.0000x reference)
import functools

import jax
import jax.numpy as jnp
from jax import lax
from jax.experimental import pallas as pl
from jax.experimental.pallas import tpu as pltpu
from jax.experimental.pallas import tpu_sc as plsc

N_USERS = 1000000
N_ITEMS = 1000000
EMB = 64
BATCH = 16384

NC = 2
NS = 16
L = 16
NW = NC * NS
B_PER_W = BATCH // NW
CHUNK = 128
NCHUNK = B_PER_W // CHUNK
GROUPS = B_PER_W // L


def _rsqrt(x):
    i = plsc.bitcast(x, jnp.int32)
    i = jnp.int32(0x5F3759DF) - lax.shift_right_logical(i, 1)
    y = plsc.bitcast(i, jnp.float32)
    for _ in range(3):
        y = y * (1.5 - 0.5 * x * y * y)
    return y


def _sc_kernel(user_hbm, item_hbm, uema_hbm, iema_hbm, u_hbm, i_hbm,
               out_hbm, idx_v, a_rows, b_rows, out_v, sem):
    wid = lax.axis_index("s") * NC + lax.axis_index("c")
    base = wid * B_PER_W

    for table_a, table_b, idx_hbm, first in (
        (user_hbm, uema_hbm, u_hbm, True),
        (item_hbm, iema_hbm, i_hbm, False),
    ):
        for j in range(NCHUNK):
            pltpu.sync_copy(idx_hbm.at[pl.ds(base + j * CHUNK, CHUNK)],
                            idx_v.at[j])
        copies = []
        for j in range(NCHUNK):
            sl = pl.ds(j * CHUNK, CHUNK)
            copies.append(pltpu.async_copy(table_a.at[idx_v.at[j]],
                                           a_rows.at[sl], sem))
            copies.append(pltpu.async_copy(table_b.at[idx_v.at[j]],
                                           b_rows.at[sl], sem))
        for c in copies:
            c.wait()

        def group_body(g, carry):
            rid = lax.iota(jnp.int32, L) + g * L
            dot = jnp.zeros((L,), jnp.float32)
            na = jnp.zeros((L,), jnp.float32)
            nb = jnp.zeros((L,), jnp.float32)
            for c in range(EMB):
                cv = jnp.full((L,), c, jnp.int32)
                va = plsc.load_gather(a_rows, [rid, cv])
                vb = plsc.load_gather(b_rows, [rid, cv])
                dot = dot + va * vb
                na = na + va * va
                nb = nb + vb * vb
            d2 = jnp.maximum(na * nb, jnp.float32(1e-16))
            cos = dot * _rsqrt(d2)
            v = (cos + 1.0) * 0.5
            sl = pl.ds(g * L, L)
            if first:
                out_v[sl] = v
            else:
                out_v[sl] = out_v[sl] + v
            return carry

        lax.fori_loop(0, GROUPS, group_body, 0)

    pltpu.sync_copy(out_v, out_hbm.at[pl.ds(base, B_PER_W)])


@jax.jit
def kernel(user_emb, item_emb, u_emb_ema, i_emb_ema, u, i):
    u = u.astype(jnp.int32)
    i = i.astype(jnp.int32)
    mesh = plsc.VectorSubcoreMesh(core_axis_name="c", subcore_axis_name="s")
    run = pl.kernel(
        _sc_kernel,
        out_type=jax.ShapeDtypeStruct((BATCH,), jnp.float32),
        mesh=mesh,
        compiler_params=pltpu.CompilerParams(needs_layout_passes=False,
                                             use_tc_tiling_on_sc=False),
        scratch_types=[
            pltpu.VMEM((NCHUNK, CHUNK), jnp.int32),
            pltpu.VMEM((B_PER_W, EMB), jnp.float32),
            pltpu.VMEM((B_PER_W, EMB), jnp.float32),
            pltpu.VMEM((B_PER_W,), jnp.float32),
            pltpu.SemaphoreType.DMA,
        ],
    )
    return run(user_emb, item_emb, u_emb_ema, i_emb_ema, u, i)


# speedup vs baseline: 2.8717x; 2.8717x over previous
import jax
import jax.numpy as jnp
from jax import lax
from jax.experimental import pallas as pl
from jax.experimental.pallas import tpu as pltpu
from jax.experimental.pallas import tpu_sc as plsc

N_USERS = 1000000
N_ITEMS = 1000000
EMB = 64
BATCH = 16384

BLK = 1024
GRID = 977
N_PAD = GRID * BLK
QROWS = N_PAD // 128

NC = 2
NS = 16
L = 16
NW = NC * NS
B_PER_W = BATCH // NW
CHUNK = 128
NCHUNK = B_PER_W // CHUNK


def _tc_dense(ut_ref, uet_ref, it_ref, iet_ref, vu_ref, vi_ref):
    def pair(a_ref, b_ref):
        a = a_ref[...]
        b = b_ref[...]
        dot = jnp.sum(a * b, axis=0)
        na = jnp.sum(a * a, axis=0)
        nb = jnp.sum(b * b, axis=0)
        cos = dot * lax.rsqrt(jnp.maximum(na * nb, jnp.float32(1e-16)))
        return (cos + 1.0) * 0.5

    vu_ref[...] = pair(ut_ref, uet_ref).reshape(8, 128)
    vi_ref[...] = pair(it_ref, iet_ref).reshape(8, 128)


def _sc_gather(vu_hbm, vi_hbm, u_hbm, i_hbm, out_hbm,
               um_v, im_v, uq_v, iq_v, ru, ri, out_v, sem):
    wid = lax.axis_index("s") * NC + lax.axis_index("c")
    base = wid * B_PER_W

    for j in range(NCHUNK):
        pltpu.sync_copy(u_hbm.at[pl.ds(base + j * CHUNK, CHUNK)], um_v.at[j])
        pltpu.sync_copy(i_hbm.at[pl.ds(base + j * CHUNK, CHUNK)], im_v.at[j])
    for j in range(NCHUNK):
        for o in range(CHUNK // L):
            sl = pl.ds(o * L, L)
            uq_v[j, sl] = lax.shift_right_logical(um_v[j, sl], 7)
            iq_v[j, sl] = lax.shift_right_logical(im_v[j, sl], 7)

    for j in range(NCHUNK):
        cu = pltpu.async_copy(vu_hbm.at[uq_v.at[j]], ru, sem)
        ci = pltpu.async_copy(vi_hbm.at[iq_v.at[j]], ri, sem)
        cu.wait()
        ci.wait()
        for g in range(CHUNK // L):
            sl = pl.ds(g * L, L)
            rows = lax.iota(jnp.int32, L) + g * L
            mu = jnp.bitwise_and(um_v[j, sl], 127)
            mi = jnp.bitwise_and(im_v[j, sl], 127)
            vmu = plsc.load_gather(ru, [rows, mu])
            vmi = plsc.load_gather(ri, [rows, mi])
            out_v[pl.ds(j * CHUNK + g * L, L)] = vmu + vmi

    pltpu.sync_copy(out_v, out_hbm.at[pl.ds(base, B_PER_W)])


@jax.jit
def kernel(user_emb, item_emb, u_emb_ema, i_emb_ema, u, i):
    u = u.astype(jnp.int32)
    i = i.astype(jnp.int32)

    tc = pl.pallas_call(
        _tc_dense,
        grid=(GRID,),
        in_specs=[pl.BlockSpec((EMB, BLK), lambda g: (0, g))] * 4,
        out_specs=[pl.BlockSpec((8, 128), lambda g: (g, 0))] * 2,
        out_shape=[jax.ShapeDtypeStruct((QROWS, 128), jnp.float32)] * 2,
    )
    vu2, vi2 = tc(user_emb.T, u_emb_ema.T, item_emb.T, i_emb_ema.T)

    mesh = plsc.VectorSubcoreMesh(core_axis_name="c", subcore_axis_name="s")
    sc = pl.kernel(
        _sc_gather,
        out_type=jax.ShapeDtypeStruct((BATCH,), jnp.float32),
        mesh=mesh,
        compiler_params=pltpu.CompilerParams(needs_layout_passes=False),
        scratch_types=[
            pltpu.VMEM((NCHUNK, CHUNK), jnp.int32),
            pltpu.VMEM((NCHUNK, CHUNK), jnp.int32),
            pltpu.VMEM((NCHUNK, CHUNK), jnp.int32),
            pltpu.VMEM((NCHUNK, CHUNK), jnp.int32),
            pltpu.VMEM((CHUNK, 128), jnp.float32),
            pltpu.VMEM((CHUNK, 128), jnp.float32),
            pltpu.VMEM((B_PER_W,), jnp.float32),
            pltpu.SemaphoreType.DMA,
        ],
    )
    return sc(vu2, vi2, u, i)


# speedup vs baseline: 5.1448x; 1.7916x over previous
import jax
import jax.numpy as jnp
from jax import lax
from jax.experimental import pallas as pl
from jax.experimental.pallas import tpu as pltpu
from jax.experimental.pallas import tpu_sc as plsc

N_USERS = 1000000
N_ITEMS = 1000000
EMB = 64
BATCH = 16384

BLK = 4096
GRID = 245
N_PAD = GRID * BLK
QROWS = N_PAD // 128

NC = 2
NS = 16
L = 16
NW = NC * NS
B_PER_W = BATCH // NW
CHUNK = 128
NCHUNK = B_PER_W // CHUNK


def _tc_dense(ut_ref, uet_ref, it_ref, iet_ref, vu_ref, vi_ref):
    def pair(a_ref, b_ref):
        a = a_ref[...]
        b = b_ref[...]
        dot = jnp.sum(a * b, axis=0)
        na = jnp.sum(a * a, axis=0)
        nb = jnp.sum(b * b, axis=0)
        cos = dot * lax.rsqrt(jnp.maximum(na * nb, jnp.float32(1e-16)))
        return (cos + 1.0) * 0.5

    vu_ref[...] = pair(ut_ref, uet_ref).reshape(BLK // 128, 128)
    vi_ref[...] = pair(it_ref, iet_ref).reshape(BLK // 128, 128)


def _sc_gather(vu_hbm, vi_hbm, u_hbm, i_hbm, out_hbm,
               um_v, im_v, uq_v, iq_v, ru, ri, out_v, sem):
    wid = lax.axis_index("s") * NC + lax.axis_index("c")
    base = wid * B_PER_W

    for j in range(NCHUNK):
        pltpu.sync_copy(u_hbm.at[pl.ds(base + j * CHUNK, CHUNK)], um_v.at[j])
        pltpu.sync_copy(i_hbm.at[pl.ds(base + j * CHUNK, CHUNK)], im_v.at[j])
    for j in range(NCHUNK):
        for o in range(CHUNK // L):
            sl = pl.ds(o * L, L)
            uq_v[j, sl] = lax.shift_right_logical(um_v[j, sl], 7)
            iq_v[j, sl] = lax.shift_right_logical(im_v[j, sl], 7)

    for j in range(NCHUNK):
        cu = pltpu.async_copy(vu_hbm.at[uq_v.at[j]], ru, sem)
        ci = pltpu.async_copy(vi_hbm.at[iq_v.at[j]], ri, sem)
        cu.wait()
        ci.wait()
        for g in range(CHUNK // L):
            sl = pl.ds(g * L, L)
            rows = lax.iota(jnp.int32, L) + g * L
            mu = jnp.bitwise_and(um_v[j, sl], 127)
            mi = jnp.bitwise_and(im_v[j, sl], 127)
            vmu = plsc.load_gather(ru, [rows, mu])
            vmi = plsc.load_gather(ri, [rows, mi])
            out_v[pl.ds(j * CHUNK + g * L, L)] = vmu + vmi

    pltpu.sync_copy(out_v, out_hbm.at[pl.ds(base, B_PER_W)])


@jax.jit
def kernel(user_emb, item_emb, u_emb_ema, i_emb_ema, u, i):
    u = u.astype(jnp.int32)
    i = i.astype(jnp.int32)

    tc = pl.pallas_call(
        _tc_dense,
        grid=(GRID,),
        in_specs=[pl.BlockSpec((EMB, BLK), lambda g: (0, g))] * 4,
        out_specs=[pl.BlockSpec((BLK // 128, 128), lambda g: (g, 0))] * 2,
        out_shape=[jax.ShapeDtypeStruct((QROWS, 128), jnp.float32)] * 2,
    )
    vu2, vi2 = tc(user_emb.T, u_emb_ema.T, item_emb.T, i_emb_ema.T)

    mesh = plsc.VectorSubcoreMesh(core_axis_name="c", subcore_axis_name="s")
    sc = pl.kernel(
        _sc_gather,
        out_type=jax.ShapeDtypeStruct((BATCH,), jnp.float32),
        mesh=mesh,
        compiler_params=pltpu.CompilerParams(needs_layout_passes=False),
        scratch_types=[
            pltpu.VMEM((NCHUNK, CHUNK), jnp.int32),
            pltpu.VMEM((NCHUNK, CHUNK), jnp.int32),
            pltpu.VMEM((NCHUNK, CHUNK), jnp.int32),
            pltpu.VMEM((NCHUNK, CHUNK), jnp.int32),
            pltpu.VMEM((CHUNK, 128), jnp.float32),
            pltpu.VMEM((CHUNK, 128), jnp.float32),
            pltpu.VMEM((B_PER_W,), jnp.float32),
            pltpu.SemaphoreType.DMA,
        ],
    )
    return sc(vu2, vi2, u, i)


# speedup vs baseline: 6.1032x; 1.1863x over previous
import jax
import jax.numpy as jnp
from jax import lax
from jax.experimental import pallas as pl
from jax.experimental.pallas import tpu as pltpu
from jax.experimental.pallas import tpu_sc as plsc

N_USERS = 1000000
N_ITEMS = 1000000
EMB = 64
BATCH = 16384

BLK = 8192
GRID = 123
N_PAD = GRID * BLK
QROWS = N_PAD // 128

NC = 2
NS = 16
L = 16
NW = NC * NS
B_PER_W = BATCH // NW
CHUNK = 128
NCHUNK = B_PER_W // CHUNK


def _tc_dense(ut_ref, uet_ref, it_ref, iet_ref, vu_ref, vi_ref):
    def pair(a_ref, b_ref):
        a = a_ref[...]
        b = b_ref[...]
        dot = jnp.sum(a * b, axis=0)
        na = jnp.sum(a * a, axis=0)
        nb = jnp.sum(b * b, axis=0)
        cos = dot * lax.rsqrt(jnp.maximum(na * nb, jnp.float32(1e-16)))
        return (cos + 1.0) * 0.5

    vu_ref[...] = pair(ut_ref, uet_ref).reshape(BLK // 128, 128)
    vi_ref[...] = pair(it_ref, iet_ref).reshape(BLK // 128, 128)


def _sc_gather(vu_hbm, vi_hbm, u_hbm, i_hbm, out_hbm,
               um_v, im_v, uq_v, iq_v, ru, ri, out_v, sem):
    wid = lax.axis_index("s") * NC + lax.axis_index("c")
    base = wid * B_PER_W

    for j in range(NCHUNK):
        pltpu.sync_copy(u_hbm.at[pl.ds(base + j * CHUNK, CHUNK)], um_v.at[j])
        pltpu.sync_copy(i_hbm.at[pl.ds(base + j * CHUNK, CHUNK)], im_v.at[j])
    for j in range(NCHUNK):
        for o in range(CHUNK // L):
            sl = pl.ds(o * L, L)
            uq_v[j, sl] = lax.shift_right_logical(um_v[j, sl], 7)
            iq_v[j, sl] = lax.shift_right_logical(im_v[j, sl], 7)

    for j in range(NCHUNK):
        cu = pltpu.async_copy(vu_hbm.at[uq_v.at[j]], ru, sem)
        ci = pltpu.async_copy(vi_hbm.at[iq_v.at[j]], ri, sem)
        cu.wait()
        ci.wait()
        for g in range(CHUNK // L):
            sl = pl.ds(g * L, L)
            rows = lax.iota(jnp.int32, L) + g * L
            mu = jnp.bitwise_and(um_v[j, sl], 127)
            mi = jnp.bitwise_and(im_v[j, sl], 127)
            vmu = plsc.load_gather(ru, [rows, mu])
            vmi = plsc.load_gather(ri, [rows, mi])
            out_v[pl.ds(j * CHUNK + g * L, L)] = vmu + vmi

    pltpu.sync_copy(out_v, out_hbm.at[pl.ds(base, B_PER_W)])


@jax.jit
def kernel(user_emb, item_emb, u_emb_ema, i_emb_ema, u, i):
    u = u.astype(jnp.int32)
    i = i.astype(jnp.int32)

    tc = pl.pallas_call(
        _tc_dense,
        grid=(GRID,),
        in_specs=[pl.BlockSpec((EMB, BLK), lambda g: (0, g))] * 4,
        out_specs=[pl.BlockSpec((BLK // 128, 128), lambda g: (g, 0))] * 2,
        out_shape=[jax.ShapeDtypeStruct((QROWS, 128), jnp.float32)] * 2,
    )
    vu2, vi2 = tc(user_emb.T, u_emb_ema.T, item_emb.T, i_emb_ema.T)

    mesh = plsc.VectorSubcoreMesh(core_axis_name="c", subcore_axis_name="s")
    sc = pl.kernel(
        _sc_gather,
        out_type=jax.ShapeDtypeStruct((BATCH,), jnp.float32),
        mesh=mesh,
        compiler_params=pltpu.CompilerParams(needs_layout_passes=False),
        scratch_types=[
            pltpu.VMEM((NCHUNK, CHUNK), jnp.int32),
            pltpu.VMEM((NCHUNK, CHUNK), jnp.int32),
            pltpu.VMEM((NCHUNK, CHUNK), jnp.int32),
            pltpu.VMEM((NCHUNK, CHUNK), jnp.int32),
            pltpu.VMEM((CHUNK, 128), jnp.float32),
            pltpu.VMEM((CHUNK, 128), jnp.float32),
            pltpu.VMEM((B_PER_W,), jnp.float32),
            pltpu.SemaphoreType.DMA,
        ],
    )
    return sc(vu2, vi2, u, i)


# speedup vs baseline: 6.4436x; 1.0558x over previous
import jax
import jax.numpy as jnp
from jax import lax
from jax.experimental import pallas as pl
from jax.experimental.pallas import tpu as pltpu
from jax.experimental.pallas import tpu_sc as plsc

N_USERS = 1000000
N_ITEMS = 1000000
EMB = 64
BATCH = 16384

BLK = 16384
GRID = 62
N_PAD = GRID * BLK
QROWS = N_PAD // 128

NC = 2
NS = 16
L = 16
NW = NC * NS
B_PER_W = BATCH // NW
CHUNK = 128
NCHUNK = B_PER_W // CHUNK


def _tc_dense(ut_ref, uet_ref, it_ref, iet_ref, vu_ref, vi_ref):
    def pair(a_ref, b_ref):
        a = a_ref[...]
        b = b_ref[...]
        dot = jnp.sum(a * b, axis=0)
        na = jnp.sum(a * a, axis=0)
        nb = jnp.sum(b * b, axis=0)
        cos = dot * lax.rsqrt(jnp.maximum(na * nb, jnp.float32(1e-16)))
        return (cos + 1.0) * 0.5

    vu_ref[...] = pair(ut_ref, uet_ref).reshape(BLK // 128, 128)
    vi_ref[...] = pair(it_ref, iet_ref).reshape(BLK // 128, 128)


def _sc_gather(vu_hbm, vi_hbm, u_hbm, i_hbm, out_hbm,
               um_v, im_v, uq_v, iq_v, ru, ri, out_v, sem):
    wid = lax.axis_index("s") * NC + lax.axis_index("c")
    base = wid * B_PER_W

    for j in range(NCHUNK):
        pltpu.sync_copy(u_hbm.at[pl.ds(base + j * CHUNK, CHUNK)], um_v.at[j])
        pltpu.sync_copy(i_hbm.at[pl.ds(base + j * CHUNK, CHUNK)], im_v.at[j])
    for j in range(NCHUNK):
        for o in range(CHUNK // L):
            sl = pl.ds(o * L, L)
            uq_v[j, sl] = lax.shift_right_logical(um_v[j, sl], 7)
            iq_v[j, sl] = lax.shift_right_logical(im_v[j, sl], 7)

    for j in range(NCHUNK):
        cu = pltpu.async_copy(vu_hbm.at[uq_v.at[j]], ru, sem)
        ci = pltpu.async_copy(vi_hbm.at[iq_v.at[j]], ri, sem)
        cu.wait()
        ci.wait()
        for g in range(CHUNK // L):
            sl = pl.ds(g * L, L)
            rows = lax.iota(jnp.int32, L) + g * L
            mu = jnp.bitwise_and(um_v[j, sl], 127)
            mi = jnp.bitwise_and(im_v[j, sl], 127)
            vmu = plsc.load_gather(ru, [rows, mu])
            vmi = plsc.load_gather(ri, [rows, mi])
            out_v[pl.ds(j * CHUNK + g * L, L)] = vmu + vmi

    pltpu.sync_copy(out_v, out_hbm.at[pl.ds(base, B_PER_W)])


@jax.jit
def kernel(user_emb, item_emb, u_emb_ema, i_emb_ema, u, i):
    u = u.astype(jnp.int32)
    i = i.astype(jnp.int32)

    tc = pl.pallas_call(
        _tc_dense,
        grid=(GRID,),
        in_specs=[pl.BlockSpec((EMB, BLK), lambda g: (0, g))] * 4,
        out_specs=[pl.BlockSpec((BLK // 128, 128), lambda g: (g, 0))] * 2,
        out_shape=[jax.ShapeDtypeStruct((QROWS, 128), jnp.float32)] * 2,
    )
    vu2, vi2 = tc(user_emb.T, u_emb_ema.T, item_emb.T, i_emb_ema.T)

    mesh = plsc.VectorSubcoreMesh(core_axis_name="c", subcore_axis_name="s")
    sc = pl.kernel(
        _sc_gather,
        out_type=jax.ShapeDtypeStruct((BATCH,), jnp.float32),
        mesh=mesh,
        compiler_params=pltpu.CompilerParams(needs_layout_passes=False),
        scratch_types=[
            pltpu.VMEM((NCHUNK, CHUNK), jnp.int32),
            pltpu.VMEM((NCHUNK, CHUNK), jnp.int32),
            pltpu.VMEM((NCHUNK, CHUNK), jnp.int32),
            pltpu.VMEM((NCHUNK, CHUNK), jnp.int32),
            pltpu.VMEM((CHUNK, 128), jnp.float32),
            pltpu.VMEM((CHUNK, 128), jnp.float32),
            pltpu.VMEM((B_PER_W,), jnp.float32),
            pltpu.SemaphoreType.DMA,
        ],
    )
    return sc(vu2, vi2, u, i)


# speedup vs baseline: 6.4719x; 1.0044x over previous
import jax
import jax.numpy as jnp
from jax import lax
from jax.experimental import pallas as pl
from jax.experimental.pallas import tpu as pltpu
from jax.experimental.pallas import tpu_sc as plsc

N_USERS = 1000000
N_ITEMS = 1000000
EMB = 64
BATCH = 16384

BLK = 24576
GRID = 41
N_PAD = GRID * BLK
QROWS = N_PAD // 128

NC = 2
NS = 16
L = 16
NW = NC * NS
B_PER_W = BATCH // NW
CHUNK = 128
NCHUNK = B_PER_W // CHUNK


def _tc_dense(ut_ref, uet_ref, it_ref, iet_ref, vu_ref, vi_ref):
    def pair(a_ref, b_ref):
        a = a_ref[...]
        b = b_ref[...]
        dot = jnp.sum(a * b, axis=0)
        na = jnp.sum(a * a, axis=0)
        nb = jnp.sum(b * b, axis=0)
        cos = dot * lax.rsqrt(jnp.maximum(na * nb, jnp.float32(1e-16)))
        return (cos + 1.0) * 0.5

    vu_ref[...] = pair(ut_ref, uet_ref).reshape(BLK // 128, 128)
    vi_ref[...] = pair(it_ref, iet_ref).reshape(BLK // 128, 128)


def _sc_gather(vu_hbm, vi_hbm, u_hbm, i_hbm, out_hbm,
               um_v, im_v, uq_v, iq_v, ru, ri, out_v, sem):
    wid = lax.axis_index("s") * NC + lax.axis_index("c")
    base = wid * B_PER_W

    for j in range(NCHUNK):
        pltpu.sync_copy(u_hbm.at[pl.ds(base + j * CHUNK, CHUNK)], um_v.at[j])
        pltpu.sync_copy(i_hbm.at[pl.ds(base + j * CHUNK, CHUNK)], im_v.at[j])
    for j in range(NCHUNK):
        for o in range(CHUNK // L):
            sl = pl.ds(o * L, L)
            uq_v[j, sl] = lax.shift_right_logical(um_v[j, sl], 7)
            iq_v[j, sl] = lax.shift_right_logical(im_v[j, sl], 7)

    for j in range(NCHUNK):
        cu = pltpu.async_copy(vu_hbm.at[uq_v.at[j]], ru, sem)
        ci = pltpu.async_copy(vi_hbm.at[iq_v.at[j]], ri, sem)
        cu.wait()
        ci.wait()
        for g in range(CHUNK // L):
            sl = pl.ds(g * L, L)
            rows = lax.iota(jnp.int32, L) + g * L
            mu = jnp.bitwise_and(um_v[j, sl], 127)
            mi = jnp.bitwise_and(im_v[j, sl], 127)
            vmu = plsc.load_gather(ru, [rows, mu])
            vmi = plsc.load_gather(ri, [rows, mi])
            out_v[pl.ds(j * CHUNK + g * L, L)] = vmu + vmi

    pltpu.sync_copy(out_v, out_hbm.at[pl.ds(base, B_PER_W)])


@jax.jit
def kernel(user_emb, item_emb, u_emb_ema, i_emb_ema, u, i):
    u = u.astype(jnp.int32)
    i = i.astype(jnp.int32)

    tc = pl.pallas_call(
        _tc_dense,
        grid=(GRID,),
        in_specs=[pl.BlockSpec((EMB, BLK), lambda g: (0, g))] * 4,
        out_specs=[pl.BlockSpec((BLK // 128, 128), lambda g: (g, 0))] * 2,
        out_shape=[jax.ShapeDtypeStruct((QROWS, 128), jnp.float32)] * 2,
    )
    vu2, vi2 = tc(user_emb.T, u_emb_ema.T, item_emb.T, i_emb_ema.T)

    mesh = plsc.VectorSubcoreMesh(core_axis_name="c", subcore_axis_name="s")
    sc = pl.kernel(
        _sc_gather,
        out_type=jax.ShapeDtypeStruct((BATCH,), jnp.float32),
        mesh=mesh,
        compiler_params=pltpu.CompilerParams(needs_layout_passes=False),
        scratch_types=[
            pltpu.VMEM((NCHUNK, CHUNK), jnp.int32),
            pltpu.VMEM((NCHUNK, CHUNK), jnp.int32),
            pltpu.VMEM((NCHUNK, CHUNK), jnp.int32),
            pltpu.VMEM((NCHUNK, CHUNK), jnp.int32),
            pltpu.VMEM((CHUNK, 128), jnp.float32),
            pltpu.VMEM((CHUNK, 128), jnp.float32),
            pltpu.VMEM((B_PER_W,), jnp.float32),
            pltpu.SemaphoreType.DMA,
        ],
    )
    return sc(vu2, vi2, u, i)
